# baseline (speedup 1.0000x reference)
import functools

import jax
import jax.numpy as jnp
from jax import lax
from jax.experimental import pallas as pl
from jax.experimental.pallas import tpu as pltpu
from jax.experimental.pallas import tpu_sc as plsc

N_LAYER, B, H, S, D = 4, 8, 8, 2048, 128
ROWS = B * H * S
NW = 32
RPW = ROWS // NW
BH_PER_W = (B * H) // NW
WIN = 8


def _sc_body(kc, vc, kval, vval, params_h, k_out, v_out,
             pbuf, wbuf, rbuf, bsem, wsem):
    w = lax.axis_index("s") * 2 + lax.axis_index("c")
    pltpu.sync_copy(params_h, pbuf)
    pvec = pbuf[...]
    layer_base = pl.multiple_of(pvec[0], 8)
    pos = pvec[1]
    pos8 = pl.multiple_of((pos // WIN) * WIN, 8)
    pos_in_win = pos % WIN
    base = w * RPW

    tensors = ((0, kc, k_out, kval), (1, vc, v_out, vval))

    bulk = []
    for t, src, dst, val in tensors:
        bulk.append(pltpu.make_async_copy(
            src.at[pl.ds(layer_base + base, RPW), :],
            dst.at[pl.ds(base, RPW), :], bsem.at[t]))
        bulk[-1].start()

    pltpu.sync_copy(kval.at[pl.ds(w * BH_PER_W, BH_PER_W), :], rbuf.at[0])
    pltpu.sync_copy(vval.at[pl.ds(w * BH_PER_W, BH_PER_W), :], rbuf.at[1])
    for t, src, dst, val in tensors:
        for j in range(BH_PER_W):
            row0 = (w * BH_PER_W + j) * S + pos8
            pltpu.sync_copy(src.at[pl.ds(layer_base + row0, WIN), :],
                            wbuf.at[t, j])
            for k in range(D // 16):
                wbuf[t, j, pos_in_win, pl.ds(16 * k, 16)] = \
                    rbuf[t, j, pl.ds(16 * k, 16)]

    patch = []
    for t, src, dst, val in tensors:
        bulk[t].wait()
        for j in range(BH_PER_W):
            row0 = (w * BH_PER_W + j) * S + pos8
            patch.append(pltpu.make_async_copy(
                wbuf.at[t, j], dst.at[pl.ds(row0, WIN), :],
                wsem.at[t * BH_PER_W + j]))
            patch[-1].start()
    for c in patch:
        c.wait()


@jax.jit
def _update(kc2, vc2, kval2, vval2, params):
    f = pl.kernel(
        _sc_body,
        out_type=(jax.ShapeDtypeStruct((ROWS, D), jnp.float32),
                  jax.ShapeDtypeStruct((ROWS, D), jnp.float32)),
        mesh=plsc.VectorSubcoreMesh(core_axis_name="c", subcore_axis_name="s"),
        scratch_types=(
            pltpu.VMEM((16,), jnp.int32),
            pltpu.VMEM((2, BH_PER_W, WIN, D), jnp.float32),
            pltpu.VMEM((2, BH_PER_W, D), jnp.float32),
            pltpu.SemaphoreType.DMA((2,)),
            pltpu.SemaphoreType.DMA((2 * BH_PER_W,)),
        ),
    )
    return f(kc2, vc2, kval2, vval2, params)


def kernel(k_cache, v_cache, layer_idx, input_pos, k_val, v_val):
    layer_idx = jnp.asarray(layer_idx, jnp.int32)
    input_pos = jnp.asarray(input_pos, jnp.int32)
    kc2 = k_cache.reshape(N_LAYER * ROWS, D)
    vc2 = v_cache.reshape(N_LAYER * ROWS, D)
    kval2 = k_val.reshape(B * H, D)
    vval2 = v_val.reshape(B * H, D)
    params = jnp.zeros((16,), jnp.int32)
    params = params.at[0].set(layer_idx * ROWS).at[1].set(input_pos)
    k2, v2 = _update(kc2, vc2, kval2, vval2, params)
    return (k2.reshape(B, H, S, D), v2.reshape(B, H, S, D))


# speedup vs baseline: 35.6762x; 35.6762x over previous
import functools

import jax
import jax.numpy as jnp
from jax import lax
from jax.experimental import pallas as pl
from jax.experimental.pallas import tpu as pltpu
from jax.experimental.pallas import tpu_sc as plsc

N_LAYER, B, H, S, D = 4, 8, 8, 2048, 128
ROWS = B * H * S
NW = 32
RPW = ROWS // NW
CH = 256
NCH = RPW // CH
NBUF = 2
BH_PER_W = (B * H) // NW


def _sc_body(kc, vc, kval, vval, params_h, k_out, v_out,
             pbuf, bufs, rbuf, gsems, ssems, psem, rsem):
    w = lax.axis_index("s") * 2 + lax.axis_index("c")

    pc = pltpu.make_async_copy(params_h, pbuf, psem)
    pc.start()
    rk = pltpu.make_async_copy(
        kval.at[pl.ds(w * BH_PER_W, BH_PER_W), :], rbuf.at[0], rsem.at[0])
    rk.start()
    rv = pltpu.make_async_copy(
        vval.at[pl.ds(w * BH_PER_W, BH_PER_W), :], rbuf.at[1], rsem.at[1])
    rv.start()
    pc.wait()
    pvec = pbuf[...]
    layer_base = pl.multiple_of(pvec[0], 8)
    pos = pvec[1]
    pos_div = pos // CH
    pos_mod = pos % CH
    base = w * RPW
    rk.wait()
    rv.wait()
    vrows = [[[rbuf[t, j, pl.ds(16 * k, 16)] for k in range(D // 16)]
              for j in range(BH_PER_W)]
             for t in range(2)]

    for t, (src, dst) in enumerate(((kc, k_out), (vc, v_out))):
        def gather(c, slot):
            r = pl.multiple_of(base + c * CH, 8)
            return pltpu.make_async_copy(
                src.at[pl.ds(layer_base + r, CH), :], bufs.at[slot],
                gsems.at[slot])

        def scatter(c, slot):
            r = pl.multiple_of(base + c * CH, 8)
            return pltpu.make_async_copy(
                bufs.at[slot], dst.at[pl.ds(r, CH), :], ssems.at[slot])

        gather(0, 0).start()

        def group(g, _):
            for b in range(NBUF):
                i = g * NBUF + b
                gather(i, b).wait()
                for j in range(BH_PER_W):
                    @pl.when(i == j * (S // CH) + pos_div)
                    def _():
                        for k in range(D // 16):
                            bufs[b, pos_mod, pl.ds(16 * k, 16)] = \
                                vrows[t][j][k]
                scatter(i, b).start()
                nb = (b + 1) % NBUF

                @pl.when(i + 1 < NCH)
                def _():
                    @pl.when(i >= 1)
                    def _():
                        scatter(i - 1, nb).wait()
                    gather(i + 1, nb).start()
            return None

        lax.fori_loop(0, NCH // NBUF, group, None, unroll=False)
        scatter(NCH - 2, (NCH - 2) % NBUF).wait()
        scatter(NCH - 1, (NCH - 1) % NBUF).wait()


@jax.jit
def _update(kc2, vc2, kval2, vval2, params):
    f = pl.kernel(
        _sc_body,
        out_type=(jax.ShapeDtypeStruct((ROWS, D), jnp.float32),
                  jax.ShapeDtypeStruct((ROWS, D), jnp.float32)),
        mesh=plsc.VectorSubcoreMesh(core_axis_name="c", subcore_axis_name="s"),
        scratch_types=(
            pltpu.VMEM((16,), jnp.int32),
            pltpu.VMEM((NBUF, CH, D), jnp.float32),
            pltpu.VMEM((2, BH_PER_W, D), jnp.float32),
            pltpu.SemaphoreType.DMA((NBUF,)),
            pltpu.SemaphoreType.DMA((NBUF,)),
            pltpu.SemaphoreType.DMA,
            pltpu.SemaphoreType.DMA((2,)),
        ),
    )
    return f(kc2, vc2, kval2, vval2, params)


def kernel(k_cache, v_cache, layer_idx, input_pos, k_val, v_val):
    layer_idx = jnp.asarray(layer_idx, jnp.int32)
    input_pos = jnp.asarray(input_pos, jnp.int32)
    kc2 = k_cache.reshape(N_LAYER * ROWS, D)
    vc2 = v_cache.reshape(N_LAYER * ROWS, D)
    kval2 = k_val.reshape(B * H, D)
    vval2 = v_val.reshape(B * H, D)
    params = jnp.zeros((16,), jnp.int32)
    params = params.at[0].set(layer_idx * ROWS).at[1].set(input_pos)
    k2, v2 = _update(kc2, vc2, kval2, vval2, params)
    return (k2.reshape(B, H, S, D), v2.reshape(B, H, S, D))


# speedup vs baseline: 36.1769x; 1.0140x over previous
import functools

import jax
import jax.numpy as jnp
from jax import lax
from jax.experimental import pallas as pl
from jax.experimental.pallas import tpu as pltpu
from jax.experimental.pallas import tpu_sc as plsc

N_LAYER, B, H, S, D = 4, 8, 8, 2048, 128
ROWS = B * H * S
NW = 32
RPW = ROWS // NW
CH = 128
NCH = RPW // CH
NBUF = 2
BH_PER_W = (B * H) // NW


def _sc_body(kc, vc, kval, vval, params_h, k_out, v_out,
             pbuf, bufs, rbuf, gsems, ssems, psem, rsem):
    w = lax.axis_index("s") * 2 + lax.axis_index("c")

    pc = pltpu.make_async_copy(params_h, pbuf, psem)
    pc.start()
    rk = pltpu.make_async_copy(
        kval.at[pl.ds(w * BH_PER_W, BH_PER_W), :], rbuf.at[0], rsem.at[0])
    rk.start()
    rv = pltpu.make_async_copy(
        vval.at[pl.ds(w * BH_PER_W, BH_PER_W), :], rbuf.at[1], rsem.at[1])
    rv.start()
    pc.wait()
    pvec = pbuf[...]
    layer_base = pl.multiple_of(pvec[0], 8)
    pos = pvec[1]
    pos_div = pos // CH
    pos_mod = pos % CH
    base = w * RPW
    rk.wait()
    rv.wait()
    vrows = [[[rbuf[t, j, pl.ds(16 * k, 16)] for k in range(D // 16)]
              for j in range(BH_PER_W)]
             for t in range(2)]

    tensors = ((kc, k_out), (vc, v_out))

    def gather(t, c, slot):
        r = pl.multiple_of(base + c * CH, 8)
        return pltpu.make_async_copy(
            tensors[t][0].at[pl.ds(layer_base + r, CH), :],
            bufs.at[t * NBUF + slot], gsems.at[t * NBUF + slot])

    def scatter(t, c, slot):
        r = pl.multiple_of(base + c * CH, 8)
        return pltpu.make_async_copy(
            bufs.at[t * NBUF + slot], tensors[t][1].at[pl.ds(r, CH), :],
            ssems.at[t * NBUF + slot])

    for t in range(2):
        gather(t, 0, 0).start()

    def group(g, _):
        for b in range(NBUF):
            i = g * NBUF + b
            for t in range(2):
                gather(t, i, b).wait()
                for j in range(BH_PER_W):
                    @pl.when(i == j * (S // CH) + pos_div)
                    def _():
                        for k in range(D // 16):
                            bufs[t * NBUF + b, pos_mod, pl.ds(16 * k, 16)] \
                                = vrows[t][j][k]
                scatter(t, i, b).start()
                nb = (b + 1) % NBUF

                @pl.when(i + 1 < NCH)
                def _():
                    @pl.when(i >= 1)
                    def _():
                        scatter(t, i - 1, nb).wait()
                    gather(t, i + 1, nb).start()
        return None

    lax.fori_loop(0, NCH // NBUF, group, None, unroll=False)
    for t in range(2):
        scatter(t, NCH - 2, (NCH - 2) % NBUF).wait()
        scatter(t, NCH - 1, (NCH - 1) % NBUF).wait()


@jax.jit
def _update(kc2, vc2, kval2, vval2, params):
    f = pl.kernel(
        _sc_body,
        out_type=(jax.ShapeDtypeStruct((ROWS, D), jnp.float32),
                  jax.ShapeDtypeStruct((ROWS, D), jnp.float32)),
        mesh=plsc.VectorSubcoreMesh(core_axis_name="c", subcore_axis_name="s"),
        scratch_types=(
            pltpu.VMEM((16,), jnp.int32),
            pltpu.VMEM((2 * NBUF, CH, D), jnp.float32),
            pltpu.VMEM((2, BH_PER_W, D), jnp.float32),
            pltpu.SemaphoreType.DMA((2 * NBUF,)),
            pltpu.SemaphoreType.DMA((2 * NBUF,)),
            pltpu.SemaphoreType.DMA,
            pltpu.SemaphoreType.DMA((2,)),
        ),
    )
    return f(kc2, vc2, kval2, vval2, params)


def kernel(k_cache, v_cache, layer_idx, input_pos, k_val, v_val):
    layer_idx = jnp.asarray(layer_idx, jnp.int32)
    input_pos = jnp.asarray(input_pos, jnp.int32)
    kc2 = k_cache.reshape(N_LAYER * ROWS, D)
    vc2 = v_cache.reshape(N_LAYER * ROWS, D)
    kval2 = k_val.reshape(B * H, D)
    vval2 = v_val.reshape(B * H, D)
    params = jnp.zeros((16,), jnp.int32)
    params = params.at[0].set(layer_idx * ROWS).at[1].set(input_pos)
    k2, v2 = _update(kc2, vc2, kval2, vval2, params)
    return (k2.reshape(B, H, S, D), v2.reshape(B, H, S, D))


# speedup vs baseline: 37.2184x; 1.0288x over previous
import functools

import jax
import jax.numpy as jnp
from jax import lax
from jax.experimental import pallas as pl
from jax.experimental.pallas import tpu as pltpu
from jax.experimental.pallas import tpu_sc as plsc

N_LAYER, B, H, S, D = 4, 8, 8, 2048, 128
ROWS = B * H * S
NW = 32
RPW = ROWS // NW
CH = 128
NCH = RPW // CH
NBUF = 2
BH_PER_W = (B * H) // NW


def _sc_body(kc, vc, kval, vval, params_h, k_out, v_out,
             pbuf, bufs, vsh, rbuf, gsems, ssems, psem, rsem):
    sid = lax.axis_index("s")
    w = sid * 2 + lax.axis_index("c")

    pc = pltpu.make_async_copy(params_h, pbuf, psem)
    pc.start()
    rk = pltpu.make_async_copy(
        kval.at[pl.ds(w * BH_PER_W, BH_PER_W), :], rbuf.at[0], rsem.at[0])
    rk.start()
    rv = pltpu.make_async_copy(
        vval.at[pl.ds(w * BH_PER_W, BH_PER_W), :], rbuf.at[1], rsem.at[1])
    rv.start()
    pc.wait()
    pvec = pbuf[...]
    layer_base = pl.multiple_of(pvec[0], 8)
    pos = pvec[1]
    pos_div = pos // CH
    pos_mod = pos % CH
    base = w * RPW
    rk.wait()
    rv.wait()
    vrows = [[[rbuf[t, j, pl.ds(16 * k, 16)] for k in range(D // 16)]
              for j in range(BH_PER_W)]
             for t in range(2)]

    tensors = ((kc, k_out), (vc, v_out))

    def staging(t, slot):
        if t == 0:
            return bufs.at[slot]
        return vsh.at[sid, slot]

    def gather(t, c, slot):
        r = pl.multiple_of(base + c * CH, 8)
        return pltpu.make_async_copy(
            tensors[t][0].at[pl.ds(layer_base + r, CH), :],
            staging(t, slot), gsems.at[t * NBUF + slot])

    def scatter(t, c, slot):
        r = pl.multiple_of(base + c * CH, 8)
        return pltpu.make_async_copy(
            staging(t, slot), tensors[t][1].at[pl.ds(r, CH), :],
            ssems.at[t * NBUF + slot])

    for t in range(2):
        gather(t, 0, 0).start()

    def group(g, _):
        for b in range(NBUF):
            i = g * NBUF + b
            for t in range(2):
                gather(t, i, b).wait()
                for j in range(BH_PER_W):
                    @pl.when(i == j * (S // CH) + pos_div)
                    def _():
                        if t == 0:
                            for k in range(D // 16):
                                bufs[b, pos_mod, pl.ds(16 * k, 16)] \
                                    = vrows[t][j][k]
                        else:
                            pltpu.sync_copy(
                                rbuf.at[1, pl.ds(j, 1), :],
                                vsh.at[sid, b, pl.ds(pos_mod, 1), :])
                scatter(t, i, b).start()
                nb = (b + 1) % NBUF

                @pl.when(i + 1 < NCH)
                def _():
                    @pl.when(i >= 1)
                    def _():
                        scatter(t, i - 1, nb).wait()
                    gather(t, i + 1, nb).start()
        return None

    lax.fori_loop(0, NCH // NBUF, group, None, unroll=False)
    for t in range(2):
        scatter(t, NCH - 2, (NCH - 2) % NBUF).wait()
        scatter(t, NCH - 1, (NCH - 1) % NBUF).wait()


@jax.jit
def _update(kc2, vc2, kval2, vval2, params):
    f = pl.kernel(
        _sc_body,
        out_type=(jax.ShapeDtypeStruct((ROWS, D), jnp.float32),
                  jax.ShapeDtypeStruct((ROWS, D), jnp.float32)),
        mesh=plsc.VectorSubcoreMesh(core_axis_name="c", subcore_axis_name="s"),
        scratch_types=(
            pltpu.VMEM((16,), jnp.int32),
            pltpu.VMEM((NBUF, CH, D), jnp.float32),
            pltpu.VMEM_SHARED((16, NBUF, CH, D), jnp.float32),
            pltpu.VMEM((2, BH_PER_W, D), jnp.float32),
            pltpu.SemaphoreType.DMA((2 * NBUF,)),
            pltpu.SemaphoreType.DMA((2 * NBUF,)),
            pltpu.SemaphoreType.DMA,
            pltpu.SemaphoreType.DMA((2,)),
        ),
    )
    return f(kc2, vc2, kval2, vval2, params)


def kernel(k_cache, v_cache, layer_idx, input_pos, k_val, v_val):
    layer_idx = jnp.asarray(layer_idx, jnp.int32)
    input_pos = jnp.asarray(input_pos, jnp.int32)
    kc2 = k_cache.reshape(N_LAYER * ROWS, D)
    vc2 = v_cache.reshape(N_LAYER * ROWS, D)
    kval2 = k_val.reshape(B * H, D)
    vval2 = v_val.reshape(B * H, D)
    params = jnp.zeros((16,), jnp.int32)
    params = params.at[0].set(layer_idx * ROWS).at[1].set(input_pos)
    k2, v2 = _update(kc2, vc2, kval2, vval2, params)
    return (k2.reshape(B, H, S, D), v2.reshape(B, H, S, D))


# speedup vs baseline: 37.3579x; 1.0037x over previous
import functools

import jax
import jax.numpy as jnp
from jax import lax
from jax.experimental import pallas as pl
from jax.experimental.pallas import tpu as pltpu
from jax.experimental.pallas import tpu_sc as plsc

N_LAYER, B, H, S, D = 4, 8, 8, 2048, 128
ROWS = B * H * S
NW = 32
RPW = ROWS // NW
CH = 128
NCH = RPW // CH
KBUF = 2
VBUF = 4
UNROLL = 4
BH_PER_W = (B * H) // NW


def _sc_body(kc, vc, kval, vval, params_h, k_out, v_out,
             pbuf, bufs, vsh, rbuf, gsems, ssems, psem, rsem):
    sid = lax.axis_index("s")
    w = sid * 2 + lax.axis_index("c")

    pc = pltpu.make_async_copy(params_h, pbuf, psem)
    pc.start()
    rk = pltpu.make_async_copy(
        kval.at[pl.ds(w * BH_PER_W, BH_PER_W), :], rbuf.at[0], rsem.at[0])
    rk.start()
    rv = pltpu.make_async_copy(
        vval.at[pl.ds(w * BH_PER_W, BH_PER_W), :], rbuf.at[1], rsem.at[1])
    rv.start()
    pc.wait()
    pvec = pbuf[...]
    layer_base = pl.multiple_of(pvec[0], 8)
    pos = pvec[1]
    pos_div = pos // CH
    pos_mod = pos % CH
    base = w * RPW
    rk.wait()
    rv.wait()
    vrows = [[[rbuf[t, j, pl.ds(16 * k, 16)] for k in range(D // 16)]
              for j in range(BH_PER_W)]
             for t in range(2)]

    tensors = ((kc, k_out), (vc, v_out))
    depth = (KBUF, VBUF)

    def staging(t, slot):
        if t == 0:
            return bufs.at[slot]
        return vsh.at[sid, slot]

    def gather(t, c, slot):
        r = pl.multiple_of(base + c * CH, 8)
        return pltpu.make_async_copy(
            tensors[t][0].at[pl.ds(layer_base + r, CH), :],
            staging(t, slot), gsems.at[t * VBUF + slot])

    def scatter(t, c, slot):
        r = pl.multiple_of(base + c * CH, 8)
        return pltpu.make_async_copy(
            staging(t, slot), tensors[t][1].at[pl.ds(r, CH), :],
            ssems.at[t * VBUF + slot])

    for t in range(2):
        for p in range(depth[t] - 1):
            gather(t, p, p).start()

    def group(g, _):
        for b in range(UNROLL):
            i = g * UNROLL + b
            for t in range(2):
                dt = depth[t]
                slot = b % dt
                gather(t, i, slot).wait()
                for j in range(BH_PER_W):
                    @pl.when(i == j * (S // CH) + pos_div)
                    def _():
                        if t == 0:
                            for k in range(D // 16):
                                bufs[slot, pos_mod, pl.ds(16 * k, 16)] \
                                    = vrows[t][j][k]
                        else:
                            pltpu.sync_copy(
                                rbuf.at[1, pl.ds(j, 1), :],
                                vsh.at[sid, slot, pl.ds(pos_mod, 1), :])
                scatter(t, i, slot).start()
                ns = (b + dt - 1) % dt

                @pl.when(i + dt - 1 < NCH)
                def _():
                    @pl.when(i >= 1)
                    def _():
                        scatter(t, i - 1, ns).wait()
                    gather(t, i + dt - 1, ns).start()
        return None

    lax.fori_loop(0, NCH // UNROLL, group, None, unroll=False)
    for t in range(2):
        for i in range(NCH - depth[t], NCH):
            scatter(t, i, i % depth[t]).wait()


@jax.jit
def _update(kc2, vc2, kval2, vval2, params):
    f = pl.kernel(
        _sc_body,
        out_type=(jax.ShapeDtypeStruct((ROWS, D), jnp.float32),
                  jax.ShapeDtypeStruct((ROWS, D), jnp.float32)),
        mesh=plsc.VectorSubcoreMesh(core_axis_name="c", subcore_axis_name="s"),
        scratch_types=(
            pltpu.VMEM((16,), jnp.int32),
            pltpu.VMEM((KBUF, CH, D), jnp.float32),
            pltpu.VMEM_SHARED((16, VBUF, CH, D), jnp.float32),
            pltpu.VMEM((2, BH_PER_W, D), jnp.float32),
            pltpu.SemaphoreType.DMA((2 * VBUF,)),
            pltpu.SemaphoreType.DMA((2 * VBUF,)),
            pltpu.SemaphoreType.DMA,
            pltpu.SemaphoreType.DMA((2,)),
        ),
    )
    return f(kc2, vc2, kval2, vval2, params)


def kernel(k_cache, v_cache, layer_idx, input_pos, k_val, v_val):
    layer_idx = jnp.asarray(layer_idx, jnp.int32)
    input_pos = jnp.asarray(input_pos, jnp.int32)
    kc2 = k_cache.reshape(N_LAYER * ROWS, D)
    vc2 = v_cache.reshape(N_LAYER * ROWS, D)
    kval2 = k_val.reshape(B * H, D)
    vval2 = v_val.reshape(B * H, D)
    params = jnp.zeros((16,), jnp.int32)
    params = params.at[0].set(layer_idx * ROWS).at[1].set(input_pos)
    k2, v2 = _update(kc2, vc2, kval2, vval2, params)
    return (k2.reshape(B, H, S, D), v2.reshape(B, H, S, D))
